# baseline (speedup 1.0000x reference)
import functools

import jax
import jax.numpy as jnp
from jax import lax
from jax.experimental import pallas as pl
from jax.experimental.pallas import tpu as pltpu
from jax.experimental.pallas import tpu_sc as plsc

N_TOKEN = 256
ATOMS_PER_TOKEN = 4
N_ATOM = N_TOKEN * ATOMS_PER_TOKEN
C_S, C_Z, C_ATOM, C_ATOM_PAIR = 384, 128, 128, 16
EPS = 1e-5

TA = 16


def _zexp_body(zt_ref, e_ref, mi_ref, mm_ref, g_ref, b_ref, w_ref, bz_ref,
               out_ref):
    x = zt_ref[...]
    mu = jnp.mean(x, axis=-1, keepdims=True)
    xc = x - mu
    var = jnp.mean(xc * xc, axis=-1, keepdims=True)
    xn = xc * jax.lax.rsqrt(var + EPS) * g_ref[0] + b_ref[0]
    bz_col = bz_ref[0][:, None]
    mm_row = mm_ref[0][None, :]
    for t in range(TA):
        yt = jax.lax.dot_general(
            w_ref[...], xn[t], (((1,), (1,)), ((), ())),
            preferred_element_type=jnp.float32)
        ct = jax.lax.dot_general(
            yt, e_ref[...], (((1,), (0,)), ((), ())),
            preferred_element_type=jnp.float32)
        out_ref[t] = (ct + bz_col) * (mi_ref[0, 0, t] * mm_row)


def _cl_body(si_ref, cl_ref, m_ref, g_ref, b_ref, w_ref, bs_ref, out_ref):
    x = si_ref[...] * m_ref[0][:, None]
    mu = jnp.mean(x, axis=-1, keepdims=True)
    xc = x - mu
    var = jnp.mean(xc * xc, axis=-1, keepdims=True)
    xn = xc * jax.lax.rsqrt(var + EPS) * g_ref[0] + b_ref[0]
    t = jax.lax.dot_general(
        xn, w_ref[...], (((1,), (1,)), ((), ())),
        preferred_element_type=jnp.float32) + bs_ref[0]
    out_ref[...] = cl_ref[...] + jnp.repeat(t, ATOMS_PER_TOKEN, axis=0)


def _sc_add_body(nc, nw, zexp_hbm, plm_hbm, out_hbm, zbuf, pbuf):
    wid = lax.axis_index("s") * nc + lax.axis_index("c")
    toks = N_TOKEN // nw
    tok0 = wid * toks

    def token_body(i, carry):
        tok = tok0 + i
        pltpu.sync_copy(zexp_hbm.at[tok], zbuf)

        def row_body(r, carry2):
            row = tok * ATOMS_PER_TOKEN + r
            pltpu.sync_copy(plm_hbm.at[row], pbuf)

            def sub_body(r2, carry3):
                for k in range(N_ATOM // 16):
                    sl = pl.ds(k * 16, 16)
                    plsc.addupdate(pbuf.at[r2, sl], zbuf[r2, sl])
                return carry3

            lax.fori_loop(0, C_ATOM_PAIR, sub_body, 0)
            pltpu.sync_copy(pbuf, out_hbm.at[row])
            return carry2

        lax.fori_loop(0, ATOMS_PER_TOKEN, row_body, 0)
        return carry

    lax.fori_loop(0, toks, token_body, 0)


@jax.jit
def kernel(token_mask, num_atoms_per_token, cl, plm, si_trunk, zij_trunk,
           ln_s_g, ln_s_b, W_s, b_s, ln_z_g, ln_z_b, W_z, b_z):
    del num_atoms_per_token
    mask2 = token_mask.reshape(1, N_TOKEN)
    mask3 = token_mask.reshape(N_TOKEN // TA, 1, TA)
    mask_m = jnp.repeat(token_mask, ATOMS_PER_TOKEN).reshape(1, N_ATOM)
    ee = (jax.lax.broadcasted_iota(jnp.int32, (N_TOKEN, N_ATOM), 1) //
          ATOMS_PER_TOKEN ==
          jax.lax.broadcasted_iota(jnp.int32, (N_TOKEN, N_ATOM), 0)
          ).astype(jnp.float32)

    plm_t = jnp.swapaxes(plm, 1, 2)

    zexp = pl.pallas_call(
        _zexp_body,
        grid=(N_TOKEN // TA,),
        in_specs=[
            pl.BlockSpec((TA, N_TOKEN, C_Z), lambda t: (t, 0, 0)),
            pl.BlockSpec((N_TOKEN, N_ATOM), lambda t: (0, 0)),
            pl.BlockSpec((1, 1, TA), lambda t: (t, 0, 0)),
            pl.BlockSpec((1, N_ATOM), lambda t: (0, 0)),
            pl.BlockSpec((1, C_Z), lambda t: (0, 0)),
            pl.BlockSpec((1, C_Z), lambda t: (0, 0)),
            pl.BlockSpec((C_ATOM_PAIR, C_Z), lambda t: (0, 0)),
            pl.BlockSpec((1, C_ATOM_PAIR), lambda t: (0, 0)),
        ],
        out_specs=pl.BlockSpec((TA, C_ATOM_PAIR, N_ATOM), lambda t: (t, 0, 0)),
        out_shape=jax.ShapeDtypeStruct((N_TOKEN, C_ATOM_PAIR, N_ATOM),
                                       jnp.float32),
    )(zij_trunk, ee, mask3, mask_m, ln_z_g.reshape(1, -1),
      ln_z_b.reshape(1, -1), W_z, b_z.reshape(1, -1))

    info = plsc.get_sparse_core_info()
    nc, ns = info.num_cores, info.num_subcores
    nw = nc * ns
    mesh = plsc.VectorSubcoreMesh(core_axis_name="c", subcore_axis_name="s")
    sc_add = functools.partial(
        pl.kernel,
        out_type=jax.ShapeDtypeStruct(plm_t.shape, plm_t.dtype),
        mesh=mesh,
        scratch_types=[
            pltpu.VMEM((C_ATOM_PAIR, N_ATOM), jnp.float32),
            pltpu.VMEM((C_ATOM_PAIR, N_ATOM), jnp.float32),
        ],
    )(functools.partial(_sc_add_body, nc, nw))
    plm_out_t = sc_add(zexp, plm_t)
    plm_out = jnp.swapaxes(plm_out_t, 1, 2)

    cl_out = pl.pallas_call(
        _cl_body,
        in_specs=[pl.BlockSpec(x.shape) for x in
                  (si_trunk, cl, mask2, ln_s_g.reshape(1, -1),
                   ln_s_b.reshape(1, -1), W_s, b_s.reshape(1, -1))],
        out_specs=pl.BlockSpec(cl.shape),
        out_shape=jax.ShapeDtypeStruct(cl.shape, cl.dtype),
    )(si_trunk, cl, mask2, ln_s_g.reshape(1, -1), ln_s_b.reshape(1, -1),
      W_s, b_s.reshape(1, -1))

    return (cl_out, plm_out)


# speedup vs baseline: 1.0734x; 1.0734x over previous
import functools

import jax
import jax.numpy as jnp
from jax import lax
from jax.experimental import pallas as pl
from jax.experimental.pallas import tpu as pltpu
from jax.experimental.pallas import tpu_sc as plsc

N_TOKEN = 256
ATOMS_PER_TOKEN = 4
N_ATOM = N_TOKEN * ATOMS_PER_TOKEN
C_S, C_Z, C_ATOM, C_ATOM_PAIR = 384, 128, 128, 16
EPS = 1e-5

TA = 16


def _zexp_body(zt_ref, e_ref, mi_ref, mm_ref, g_ref, b_ref, w_ref, bz_ref,
               out_ref):
    x = zt_ref[...]
    mu = jnp.mean(x, axis=-1, keepdims=True)
    xc = x - mu
    var = jnp.mean(xc * xc, axis=-1, keepdims=True)
    xn = xc * jax.lax.rsqrt(var + EPS) * g_ref[0] + b_ref[0]
    bz_col = bz_ref[0][:, None]
    mm_row = mm_ref[0][None, :]
    for t in range(TA):
        yt = jax.lax.dot_general(
            w_ref[...], xn[t], (((1,), (1,)), ((), ())),
            preferred_element_type=jnp.float32)
        ct = jax.lax.dot_general(
            yt, e_ref[...], (((1,), (0,)), ((), ())),
            preferred_element_type=jnp.float32)
        out_ref[t] = (ct + bz_col) * (mi_ref[0, 0, t] * mm_row)


def _cl_body(si_ref, cl_ref, m_ref, g_ref, b_ref, w_ref, bs_ref, out_ref):
    x = si_ref[...] * m_ref[0][:, None]
    mu = jnp.mean(x, axis=-1, keepdims=True)
    xc = x - mu
    var = jnp.mean(xc * xc, axis=-1, keepdims=True)
    xn = xc * jax.lax.rsqrt(var + EPS) * g_ref[0] + b_ref[0]
    t = jax.lax.dot_general(
        xn, w_ref[...], (((1,), (1,)), ((), ())),
        preferred_element_type=jnp.float32) + bs_ref[0]
    out_ref[...] = cl_ref[...] + jnp.repeat(t, ATOMS_PER_TOKEN, axis=0)


def _sc_add_body(nc, nw, zexp_hbm, plm_hbm, out_hbm, zbuf, pbuf):
    wid = lax.axis_index("s") * nc + lax.axis_index("c")
    toks = N_TOKEN // nw
    tok0 = wid * toks

    def token_body(i, carry):
        tok = tok0 + i
        row0 = tok * ATOMS_PER_TOKEN
        pltpu.sync_copy(zexp_hbm.at[tok], zbuf)
        pltpu.sync_copy(plm_hbm.at[pl.ds(row0, ATOMS_PER_TOKEN)], pbuf)

        def sub_body(q, carry3):
            r = q // C_ATOM_PAIR
            r2 = q % C_ATOM_PAIR
            for k in range(N_ATOM // 16):
                sl = pl.ds(k * 16, 16)
                plsc.addupdate(pbuf.at[r, r2, sl], zbuf[r2, sl])
            return carry3

        lax.fori_loop(0, ATOMS_PER_TOKEN * C_ATOM_PAIR, sub_body, 0)
        pltpu.sync_copy(pbuf, out_hbm.at[pl.ds(row0, ATOMS_PER_TOKEN)])
        return carry

    lax.fori_loop(0, toks, token_body, 0)


@jax.jit
def kernel(token_mask, num_atoms_per_token, cl, plm, si_trunk, zij_trunk,
           ln_s_g, ln_s_b, W_s, b_s, ln_z_g, ln_z_b, W_z, b_z):
    del num_atoms_per_token
    mask2 = token_mask.reshape(1, N_TOKEN)
    mask3 = token_mask.reshape(N_TOKEN // TA, 1, TA)
    mask_m = jnp.repeat(token_mask, ATOMS_PER_TOKEN).reshape(1, N_ATOM)
    ee = (jax.lax.broadcasted_iota(jnp.int32, (N_TOKEN, N_ATOM), 1) //
          ATOMS_PER_TOKEN ==
          jax.lax.broadcasted_iota(jnp.int32, (N_TOKEN, N_ATOM), 0)
          ).astype(jnp.float32)

    plm_t = jnp.swapaxes(plm, 1, 2)

    zexp = pl.pallas_call(
        _zexp_body,
        grid=(N_TOKEN // TA,),
        in_specs=[
            pl.BlockSpec((TA, N_TOKEN, C_Z), lambda t: (t, 0, 0)),
            pl.BlockSpec((N_TOKEN, N_ATOM), lambda t: (0, 0)),
            pl.BlockSpec((1, 1, TA), lambda t: (t, 0, 0)),
            pl.BlockSpec((1, N_ATOM), lambda t: (0, 0)),
            pl.BlockSpec((1, C_Z), lambda t: (0, 0)),
            pl.BlockSpec((1, C_Z), lambda t: (0, 0)),
            pl.BlockSpec((C_ATOM_PAIR, C_Z), lambda t: (0, 0)),
            pl.BlockSpec((1, C_ATOM_PAIR), lambda t: (0, 0)),
        ],
        out_specs=pl.BlockSpec((TA, C_ATOM_PAIR, N_ATOM), lambda t: (t, 0, 0)),
        out_shape=jax.ShapeDtypeStruct((N_TOKEN, C_ATOM_PAIR, N_ATOM),
                                       jnp.float32),
    )(zij_trunk, ee, mask3, mask_m, ln_z_g.reshape(1, -1),
      ln_z_b.reshape(1, -1), W_z, b_z.reshape(1, -1))

    info = plsc.get_sparse_core_info()
    nc, ns = info.num_cores, info.num_subcores
    nw = nc * ns
    mesh = plsc.VectorSubcoreMesh(core_axis_name="c", subcore_axis_name="s")
    sc_add = functools.partial(
        pl.kernel,
        out_type=jax.ShapeDtypeStruct(plm_t.shape, plm_t.dtype),
        mesh=mesh,
        scratch_types=[
            pltpu.VMEM((C_ATOM_PAIR, N_ATOM), jnp.float32),
            pltpu.VMEM((ATOMS_PER_TOKEN, C_ATOM_PAIR, N_ATOM), jnp.float32),
        ],
    )(functools.partial(_sc_add_body, nc, nw))
    plm_out_t = sc_add(zexp, plm_t)
    plm_out = jnp.swapaxes(plm_out_t, 1, 2)

    cl_out = pl.pallas_call(
        _cl_body,
        in_specs=[pl.BlockSpec(x.shape) for x in
                  (si_trunk, cl, mask2, ln_s_g.reshape(1, -1),
                   ln_s_b.reshape(1, -1), W_s, b_s.reshape(1, -1))],
        out_specs=pl.BlockSpec(cl.shape),
        out_shape=jax.ShapeDtypeStruct(cl.shape, cl.dtype),
    )(si_trunk, cl, mask2, ln_s_g.reshape(1, -1), ln_s_b.reshape(1, -1),
      W_s, b_s.reshape(1, -1))

    return (cl_out, plm_out)


# speedup vs baseline: 1.6645x; 1.5506x over previous
import functools

import jax
import jax.numpy as jnp
from jax import lax
from jax.experimental import pallas as pl
from jax.experimental.pallas import tpu as pltpu
from jax.experimental.pallas import tpu_sc as plsc

N_TOKEN = 256
ATOMS_PER_TOKEN = 4
N_ATOM = N_TOKEN * ATOMS_PER_TOKEN
C_S, C_Z, C_ATOM, C_ATOM_PAIR = 384, 128, 128, 16
EPS = 1e-5

TA = 16


def _zexp_body(zt_ref, e_ref, mi_ref, mm_ref, g_ref, b_ref, w_ref, bz_ref,
               out_ref):
    x = zt_ref[...]
    mu = jnp.mean(x, axis=-1, keepdims=True)
    xc = x - mu
    var = jnp.mean(xc * xc, axis=-1, keepdims=True)
    xn = xc * jax.lax.rsqrt(var + EPS) * g_ref[0] + b_ref[0]
    bz_col = bz_ref[0][:, None]
    mm_row = mm_ref[0][None, :]
    for t in range(TA):
        yt = jax.lax.dot_general(
            w_ref[...], xn[t], (((1,), (1,)), ((), ())),
            preferred_element_type=jnp.float32)
        ct = jax.lax.dot_general(
            yt, e_ref[...], (((1,), (0,)), ((), ())),
            preferred_element_type=jnp.float32)
        out_ref[t] = (ct + bz_col) * (mi_ref[0, 0, t] * mm_row)


def _cl_body(si_ref, cl_ref, m_ref, g_ref, b_ref, w_ref, bs_ref, out_ref):
    x = si_ref[...] * m_ref[0][:, None]
    mu = jnp.mean(x, axis=-1, keepdims=True)
    xc = x - mu
    var = jnp.mean(xc * xc, axis=-1, keepdims=True)
    xn = xc * jax.lax.rsqrt(var + EPS) * g_ref[0] + b_ref[0]
    t = jax.lax.dot_general(
        xn, w_ref[...], (((1,), (1,)), ((), ())),
        preferred_element_type=jnp.float32) + bs_ref[0]
    out_ref[...] = cl_ref[...] + jnp.repeat(t, ATOMS_PER_TOKEN, axis=0)


def _sc_add_body(nc, nw, zexp_hbm, plm_hbm, out_hbm, zbuf, pbuf):
    wid = lax.axis_index("s") * nc + lax.axis_index("c")
    toks = N_TOKEN // nw
    tok0 = wid * toks

    def token_body(i, carry):
        tok = tok0 + i
        row0 = tok * ATOMS_PER_TOKEN
        pltpu.sync_copy(zexp_hbm.at[tok], zbuf)
        pltpu.sync_copy(plm_hbm.at[pl.ds(row0, ATOMS_PER_TOKEN)], pbuf)

        def sub_body(q, carry3):
            r2 = q // 8
            kb = (q % 8) * 8
            sls = [pl.ds((kb + j) * 16, 16) for j in range(8)]
            vals = [zbuf[r2, sl] for sl in sls]
            for j in range(8):
                for r in range(ATOMS_PER_TOKEN):
                    plsc.addupdate(pbuf.at[r, r2, sls[j]], vals[j])
            return carry3

        lax.fori_loop(0, C_ATOM_PAIR * 8, sub_body, 0)
        pltpu.sync_copy(pbuf, out_hbm.at[pl.ds(row0, ATOMS_PER_TOKEN)])
        return carry

    lax.fori_loop(0, toks, token_body, 0)


@jax.jit
def kernel(token_mask, num_atoms_per_token, cl, plm, si_trunk, zij_trunk,
           ln_s_g, ln_s_b, W_s, b_s, ln_z_g, ln_z_b, W_z, b_z):
    del num_atoms_per_token
    mask2 = token_mask.reshape(1, N_TOKEN)
    mask3 = token_mask.reshape(N_TOKEN // TA, 1, TA)
    mask_m = jnp.repeat(token_mask, ATOMS_PER_TOKEN).reshape(1, N_ATOM)
    ee = (jax.lax.broadcasted_iota(jnp.int32, (N_TOKEN, N_ATOM), 1) //
          ATOMS_PER_TOKEN ==
          jax.lax.broadcasted_iota(jnp.int32, (N_TOKEN, N_ATOM), 0)
          ).astype(jnp.float32)

    plm_t = jnp.swapaxes(plm, 1, 2)

    zexp = pl.pallas_call(
        _zexp_body,
        grid=(N_TOKEN // TA,),
        in_specs=[
            pl.BlockSpec((TA, N_TOKEN, C_Z), lambda t: (t, 0, 0)),
            pl.BlockSpec((N_TOKEN, N_ATOM), lambda t: (0, 0)),
            pl.BlockSpec((1, 1, TA), lambda t: (t, 0, 0)),
            pl.BlockSpec((1, N_ATOM), lambda t: (0, 0)),
            pl.BlockSpec((1, C_Z), lambda t: (0, 0)),
            pl.BlockSpec((1, C_Z), lambda t: (0, 0)),
            pl.BlockSpec((C_ATOM_PAIR, C_Z), lambda t: (0, 0)),
            pl.BlockSpec((1, C_ATOM_PAIR), lambda t: (0, 0)),
        ],
        out_specs=pl.BlockSpec((TA, C_ATOM_PAIR, N_ATOM), lambda t: (t, 0, 0)),
        out_shape=jax.ShapeDtypeStruct((N_TOKEN, C_ATOM_PAIR, N_ATOM),
                                       jnp.float32),
    )(zij_trunk, ee, mask3, mask_m, ln_z_g.reshape(1, -1),
      ln_z_b.reshape(1, -1), W_z, b_z.reshape(1, -1))

    info = plsc.get_sparse_core_info()
    nc, ns = info.num_cores, info.num_subcores
    nw = nc * ns
    mesh = plsc.VectorSubcoreMesh(core_axis_name="c", subcore_axis_name="s")
    sc_add = functools.partial(
        pl.kernel,
        out_type=jax.ShapeDtypeStruct(plm_t.shape, plm_t.dtype),
        mesh=mesh,
        scratch_types=[
            pltpu.VMEM((C_ATOM_PAIR, N_ATOM), jnp.float32),
            pltpu.VMEM((ATOMS_PER_TOKEN, C_ATOM_PAIR, N_ATOM), jnp.float32),
        ],
    )(functools.partial(_sc_add_body, nc, nw))
    plm_out_t = sc_add(zexp, plm_t)
    plm_out = jnp.swapaxes(plm_out_t, 1, 2)

    cl_out = pl.pallas_call(
        _cl_body,
        in_specs=[pl.BlockSpec(x.shape) for x in
                  (si_trunk, cl, mask2, ln_s_g.reshape(1, -1),
                   ln_s_b.reshape(1, -1), W_s, b_s.reshape(1, -1))],
        out_specs=pl.BlockSpec(cl.shape),
        out_shape=jax.ShapeDtypeStruct(cl.shape, cl.dtype),
    )(si_trunk, cl, mask2, ln_s_g.reshape(1, -1), ln_s_b.reshape(1, -1),
      W_s, b_s.reshape(1, -1))

    return (cl_out, plm_out)


# speedup vs baseline: 1.9464x; 1.1694x over previous
import functools

import jax
import jax.numpy as jnp
from jax import lax
from jax.experimental import pallas as pl
from jax.experimental.pallas import tpu as pltpu
from jax.experimental.pallas import tpu_sc as plsc

N_TOKEN = 256
ATOMS_PER_TOKEN = 4
N_ATOM = N_TOKEN * ATOMS_PER_TOKEN
C_S, C_Z, C_ATOM, C_ATOM_PAIR = 384, 128, 128, 16
EPS = 1e-5

TA = 16


def _zexp_body(zt_ref, e_ref, mi_ref, mm_ref, g_ref, b_ref, w_ref, bz_ref,
               out_ref):
    x = zt_ref[...]
    mu = jnp.mean(x, axis=-1, keepdims=True)
    xc = x - mu
    var = jnp.mean(xc * xc, axis=-1, keepdims=True)
    xn = xc * jax.lax.rsqrt(var + EPS) * g_ref[0] + b_ref[0]
    bz_col = bz_ref[0][:, None]
    mm_row = mm_ref[0][None, :]
    for t in range(TA):
        yt = jax.lax.dot_general(
            w_ref[...], xn[t], (((1,), (1,)), ((), ())),
            preferred_element_type=jnp.float32)
        ct = jax.lax.dot_general(
            yt, e_ref[...], (((1,), (0,)), ((), ())),
            preferred_element_type=jnp.float32)
        out_ref[t] = (ct + bz_col) * (mi_ref[0, 0, t] * mm_row)


def _cl_body(si_ref, cl_ref, m_ref, g_ref, b_ref, w_ref, bs_ref, out_ref):
    x = si_ref[...] * m_ref[0][:, None]
    mu = jnp.mean(x, axis=-1, keepdims=True)
    xc = x - mu
    var = jnp.mean(xc * xc, axis=-1, keepdims=True)
    xn = xc * jax.lax.rsqrt(var + EPS) * g_ref[0] + b_ref[0]
    t = jax.lax.dot_general(
        xn, w_ref[...], (((1,), (1,)), ((), ())),
        preferred_element_type=jnp.float32) + bs_ref[0]
    out_ref[...] = cl_ref[...] + jnp.repeat(t, ATOMS_PER_TOKEN, axis=0)


def _sc_add_body(nc, nw, zexp_hbm, plm_hbm, out_hbm, zbuf,
                 pbuf0, pbuf1, pbuf2, si0, si1, si2, so0, so1, so2):
    wid = lax.axis_index("s") * nc + lax.axis_index("c")
    toks = N_TOKEN // nw
    tok0 = wid * toks
    row_base = tok0 * ATOMS_PER_TOKEN
    bufs = [pbuf0, pbuf1, pbuf2]
    sin = [si0, si1, si2]
    sout = [so0, so1, so2]
    nchunks = toks * 2

    def add_chunk(pbuf):
        def sub_body(q, carry):
            r2 = q // 8
            kb = (q % 8) * 8
            sls = [pl.ds((kb + j) * 16, 16) for j in range(8)]
            vals = [zbuf[r2, sl] for sl in sls]
            for j in range(8):
                for r in range(2):
                    plsc.addupdate(pbuf.at[r, r2, sls[j]], vals[j])
            return carry

        lax.fori_loop(0, C_ATOM_PAIR * 8, sub_body, 0)

    def start_in(c):
        rows = pl.ds(row_base + 2 * c, 2)
        return pltpu.async_copy(plm_hbm.at[rows], bufs[c % 3], sin[c % 3])

    pend_in = {0: start_in(0)}
    pend_out = {}
    for c in range(nchunks):
        b = c % 3
        if c + 1 < nchunks:
            nb = (c + 1) % 3
            if c - 2 >= 0 and (c - 2) in pend_out:
                pend_out.pop(c - 2).wait()
            pend_in[c + 1] = start_in(c + 1)
        if c % 2 == 0:
            pltpu.sync_copy(zexp_hbm.at[tok0 + c // 2], zbuf)
        pend_in.pop(c).wait()
        add_chunk(bufs[b])
        rows = pl.ds(row_base + 2 * c, 2)
        pend_out[c] = pltpu.async_copy(bufs[b], out_hbm.at[rows], sout[b])
    for c in sorted(pend_out):
        pend_out.pop(c).wait()


@jax.jit
def kernel(token_mask, num_atoms_per_token, cl, plm, si_trunk, zij_trunk,
           ln_s_g, ln_s_b, W_s, b_s, ln_z_g, ln_z_b, W_z, b_z):
    del num_atoms_per_token
    mask2 = token_mask.reshape(1, N_TOKEN)
    mask3 = token_mask.reshape(N_TOKEN // TA, 1, TA)
    mask_m = jnp.repeat(token_mask, ATOMS_PER_TOKEN).reshape(1, N_ATOM)
    ee = (jax.lax.broadcasted_iota(jnp.int32, (N_TOKEN, N_ATOM), 1) //
          ATOMS_PER_TOKEN ==
          jax.lax.broadcasted_iota(jnp.int32, (N_TOKEN, N_ATOM), 0)
          ).astype(jnp.float32)

    plm_t = jnp.swapaxes(plm, 1, 2)

    zexp = pl.pallas_call(
        _zexp_body,
        grid=(N_TOKEN // TA,),
        in_specs=[
            pl.BlockSpec((TA, N_TOKEN, C_Z), lambda t: (t, 0, 0)),
            pl.BlockSpec((N_TOKEN, N_ATOM), lambda t: (0, 0)),
            pl.BlockSpec((1, 1, TA), lambda t: (t, 0, 0)),
            pl.BlockSpec((1, N_ATOM), lambda t: (0, 0)),
            pl.BlockSpec((1, C_Z), lambda t: (0, 0)),
            pl.BlockSpec((1, C_Z), lambda t: (0, 0)),
            pl.BlockSpec((C_ATOM_PAIR, C_Z), lambda t: (0, 0)),
            pl.BlockSpec((1, C_ATOM_PAIR), lambda t: (0, 0)),
        ],
        out_specs=pl.BlockSpec((TA, C_ATOM_PAIR, N_ATOM), lambda t: (t, 0, 0)),
        out_shape=jax.ShapeDtypeStruct((N_TOKEN, C_ATOM_PAIR, N_ATOM),
                                       jnp.float32),
    )(zij_trunk, ee, mask3, mask_m, ln_z_g.reshape(1, -1),
      ln_z_b.reshape(1, -1), W_z, b_z.reshape(1, -1))

    info = plsc.get_sparse_core_info()
    nc, ns = info.num_cores, info.num_subcores
    nw = nc * ns
    mesh = plsc.VectorSubcoreMesh(core_axis_name="c", subcore_axis_name="s")
    sc_add = functools.partial(
        pl.kernel,
        out_type=jax.ShapeDtypeStruct(plm_t.shape, plm_t.dtype),
        mesh=mesh,
        scratch_types=(
            [pltpu.VMEM((C_ATOM_PAIR, N_ATOM), jnp.float32)] +
            [pltpu.VMEM((2, C_ATOM_PAIR, N_ATOM), jnp.float32)] * 3 +
            [pltpu.SemaphoreType.DMA] * 6
        ),
    )(functools.partial(_sc_add_body, nc, nw))
    plm_out_t = sc_add(zexp, plm_t)
    plm_out = jnp.swapaxes(plm_out_t, 1, 2)

    cl_out = pl.pallas_call(
        _cl_body,
        in_specs=[pl.BlockSpec(x.shape) for x in
                  (si_trunk, cl, mask2, ln_s_g.reshape(1, -1),
                   ln_s_b.reshape(1, -1), W_s, b_s.reshape(1, -1))],
        out_specs=pl.BlockSpec(cl.shape),
        out_shape=jax.ShapeDtypeStruct(cl.shape, cl.dtype),
    )(si_trunk, cl, mask2, ln_s_g.reshape(1, -1), ln_s_b.reshape(1, -1),
      W_s, b_s.reshape(1, -1))

    return (cl_out, plm_out)


# speedup vs baseline: 1.9533x; 1.0035x over previous
import functools

import jax
import jax.numpy as jnp
from jax import lax
from jax.experimental import pallas as pl
from jax.experimental.pallas import tpu as pltpu
from jax.experimental.pallas import tpu_sc as plsc

N_TOKEN = 256
ATOMS_PER_TOKEN = 4
N_ATOM = N_TOKEN * ATOMS_PER_TOKEN
C_S, C_Z, C_ATOM, C_ATOM_PAIR = 384, 128, 128, 16
EPS = 1e-5

TA = 32


def _zexp_body(zt_ref, e_ref, mi_ref, mm_ref, g_ref, b_ref, w_ref, bz_ref,
               out_ref):
    x = zt_ref[...]
    mu = jnp.mean(x, axis=-1, keepdims=True)
    xc = x - mu
    var = jnp.mean(xc * xc, axis=-1, keepdims=True)
    xn = xc * jax.lax.rsqrt(var + EPS) * g_ref[0] + b_ref[0]
    bz_col = bz_ref[0][:, None]
    mm_row = mm_ref[0][None, :]
    for t in range(TA):
        yt = jax.lax.dot_general(
            w_ref[...], xn[t], (((1,), (1,)), ((), ())),
            preferred_element_type=jnp.float32)
        ct = jax.lax.dot_general(
            yt, e_ref[...], (((1,), (0,)), ((), ())),
            preferred_element_type=jnp.float32)
        out_ref[t] = (ct + bz_col) * (mi_ref[0, 0, t] * mm_row)


def _cl_body(si_ref, cl_ref, m_ref, g_ref, b_ref, w_ref, bs_ref, out_ref):
    x = si_ref[...] * m_ref[0][:, None]
    mu = jnp.mean(x, axis=-1, keepdims=True)
    xc = x - mu
    var = jnp.mean(xc * xc, axis=-1, keepdims=True)
    xn = xc * jax.lax.rsqrt(var + EPS) * g_ref[0] + b_ref[0]
    t = jax.lax.dot_general(
        xn, w_ref[...], (((1,), (1,)), ((), ())),
        preferred_element_type=jnp.float32) + bs_ref[0]
    out_ref[...] = cl_ref[...] + jnp.repeat(t, ATOMS_PER_TOKEN, axis=0)


def _sc_add_body(nc, nw, zexp_hbm, plm_hbm, out_hbm, zbuf,
                 pbuf0, pbuf1, pbuf2, si0, si1, si2, so0, so1, so2):
    wid = lax.axis_index("s") * nc + lax.axis_index("c")
    toks = N_TOKEN // nw
    tok0 = wid * toks
    row_base = tok0 * ATOMS_PER_TOKEN
    bufs = [pbuf0, pbuf1, pbuf2]
    sin = [si0, si1, si2]
    sout = [so0, so1, so2]
    nchunks = toks * 2

    def add_chunk(pbuf):
        def sub_body(q, carry):
            r2 = q // 8
            kb = (q % 8) * 8
            sls = [pl.ds((kb + j) * 16, 16) for j in range(8)]
            vals = [zbuf[r2, sl] for sl in sls]
            for j in range(8):
                for r in range(2):
                    plsc.addupdate(pbuf.at[r, r2, sls[j]], vals[j])
            return carry

        lax.fori_loop(0, C_ATOM_PAIR * 8, sub_body, 0)

    def start_in(c):
        rows = pl.ds(row_base + 2 * c, 2)
        return pltpu.async_copy(plm_hbm.at[rows], bufs[c % 3], sin[c % 3])

    pend_in = {0: start_in(0)}
    pend_out = {}
    for c in range(nchunks):
        b = c % 3
        if c + 1 < nchunks:
            nb = (c + 1) % 3
            if c - 2 >= 0 and (c - 2) in pend_out:
                pend_out.pop(c - 2).wait()
            pend_in[c + 1] = start_in(c + 1)
        if c % 2 == 0:
            pltpu.sync_copy(zexp_hbm.at[tok0 + c // 2], zbuf)
        pend_in.pop(c).wait()
        add_chunk(bufs[b])
        rows = pl.ds(row_base + 2 * c, 2)
        pend_out[c] = pltpu.async_copy(bufs[b], out_hbm.at[rows], sout[b])
    for c in sorted(pend_out):
        pend_out.pop(c).wait()


@jax.jit
def kernel(token_mask, num_atoms_per_token, cl, plm, si_trunk, zij_trunk,
           ln_s_g, ln_s_b, W_s, b_s, ln_z_g, ln_z_b, W_z, b_z):
    del num_atoms_per_token
    mask2 = token_mask.reshape(1, N_TOKEN)
    mask3 = token_mask.reshape(N_TOKEN // TA, 1, TA)
    mask_m = jnp.repeat(token_mask, ATOMS_PER_TOKEN).reshape(1, N_ATOM)
    ee = (jax.lax.broadcasted_iota(jnp.int32, (N_TOKEN, N_ATOM), 1) //
          ATOMS_PER_TOKEN ==
          jax.lax.broadcasted_iota(jnp.int32, (N_TOKEN, N_ATOM), 0)
          ).astype(jnp.float32)

    plm_t = jnp.swapaxes(plm, 1, 2)

    zexp = pl.pallas_call(
        _zexp_body,
        grid=(N_TOKEN // TA,),
        in_specs=[
            pl.BlockSpec((TA, N_TOKEN, C_Z), lambda t: (t, 0, 0)),
            pl.BlockSpec((N_TOKEN, N_ATOM), lambda t: (0, 0)),
            pl.BlockSpec((1, 1, TA), lambda t: (t, 0, 0)),
            pl.BlockSpec((1, N_ATOM), lambda t: (0, 0)),
            pl.BlockSpec((1, C_Z), lambda t: (0, 0)),
            pl.BlockSpec((1, C_Z), lambda t: (0, 0)),
            pl.BlockSpec((C_ATOM_PAIR, C_Z), lambda t: (0, 0)),
            pl.BlockSpec((1, C_ATOM_PAIR), lambda t: (0, 0)),
        ],
        out_specs=pl.BlockSpec((TA, C_ATOM_PAIR, N_ATOM), lambda t: (t, 0, 0)),
        out_shape=jax.ShapeDtypeStruct((N_TOKEN, C_ATOM_PAIR, N_ATOM),
                                       jnp.float32),
    )(zij_trunk, ee, mask3, mask_m, ln_z_g.reshape(1, -1),
      ln_z_b.reshape(1, -1), W_z, b_z.reshape(1, -1))

    info = plsc.get_sparse_core_info()
    nc, ns = info.num_cores, info.num_subcores
    nw = nc * ns
    mesh = plsc.VectorSubcoreMesh(core_axis_name="c", subcore_axis_name="s")
    sc_add = functools.partial(
        pl.kernel,
        out_type=jax.ShapeDtypeStruct(plm_t.shape, plm_t.dtype),
        mesh=mesh,
        scratch_types=(
            [pltpu.VMEM((C_ATOM_PAIR, N_ATOM), jnp.float32)] +
            [pltpu.VMEM((2, C_ATOM_PAIR, N_ATOM), jnp.float32)] * 3 +
            [pltpu.SemaphoreType.DMA] * 6
        ),
    )(functools.partial(_sc_add_body, nc, nw))
    plm_out_t = sc_add(zexp, plm_t)
    plm_out = jnp.swapaxes(plm_out_t, 1, 2)

    cl_out = pl.pallas_call(
        _cl_body,
        in_specs=[pl.BlockSpec(x.shape) for x in
                  (si_trunk, cl, mask2, ln_s_g.reshape(1, -1),
                   ln_s_b.reshape(1, -1), W_s, b_s.reshape(1, -1))],
        out_specs=pl.BlockSpec(cl.shape),
        out_shape=jax.ShapeDtypeStruct(cl.shape, cl.dtype),
    )(si_trunk, cl, mask2, ln_s_g.reshape(1, -1), ln_s_b.reshape(1, -1),
      W_s, b_s.reshape(1, -1))

    return (cl_out, plm_out)
